# Initial kernel scaffold; baseline (speedup 1.0000x reference)
import jax
import jax.numpy as jnp
from jax.experimental import pallas as pl


def kernel(e, lp):
    raise NotImplementedError("write your pallas kernel here")


# speedup vs baseline: 13.8001x; 13.8001x over previous
import functools

import jax
import jax.numpy as jnp
from jax import lax
from jax.experimental import pallas as pl
from jax.experimental.pallas import tpu as pltpu

N = 4096
D = 128
K = 8
R = 256
NB = N // R


def _tc_body(e_blk_ref, e_all_ref, out_ref, sb_ref, acc_ref, refstd_ref):
    i = pl.program_id(0)

    @pl.when(i == 0)
    def _():
        ea = e_all_ref[...]
        sq = ea * ea
        ones = jnp.ones((1, D), dtype=jnp.float32)
        sb_ref[...] = lax.dot_general(
            ones, sq, (((1,), (1,)), ((), ())),
            preferred_element_type=jnp.float32)
        colsum = jnp.sum(ea, axis=0, keepdims=True)
        colsum2 = jnp.sum(sq, axis=0, keepdims=True)
        var = (colsum2 - colsum * colsum * (1.0 / N)) * (1.0 / (N - 1))
        refstd_ref[0] = jnp.sum(var) * (1.0 / D)
        acc_ref[0] = 0.0

    e_blk = e_blk_ref[...]
    sa = jnp.sum(e_blk * e_blk, axis=1, keepdims=True)
    g = lax.dot_general(
        e_blk, e_all_ref[...], (((1,), (1,)), ((), ())),
        preferred_element_type=jnp.float32)
    d2 = jnp.maximum(sa + sb_ref[...] - 2.0 * g, 0.0)

    iota = lax.broadcasted_iota(jnp.int32, (R, N), 1)
    x = d2
    ssum = jnp.zeros((R, 1), dtype=jnp.float32)
    for _ in range(K):
        m = jnp.min(x, axis=1, keepdims=True)
        ssum = ssum + m
        cand = jnp.where(x <= m, iota, N)
        first = jnp.min(cand, axis=1, keepdims=True)
        x = jnp.where(iota == first, jnp.inf, x)

    step_sum = jnp.sum(ssum)
    total = acc_ref[0] + step_sum
    acc_ref[0] = total

    @pl.when(i == NB - 1)
    def _():
        out_ref[0, 0] = total * (1.0 / (N * K)) / refstd_ref[0]


def kernel(e, lp):
    del lp
    out = pl.pallas_call(
        _tc_body,
        grid=(NB,),
        in_specs=[
            pl.BlockSpec((R, D), lambda i: (i, 0)),
            pl.BlockSpec((N, D), lambda i: (0, 0)),
        ],
        out_specs=pl.BlockSpec(memory_space=pltpu.SMEM),
        out_shape=jax.ShapeDtypeStruct((1, 1), jnp.float32),
        scratch_shapes=[
            pltpu.VMEM((1, N), jnp.float32),
            pltpu.SMEM((1,), jnp.float32),
            pltpu.SMEM((1,), jnp.float32),
        ],
    )(e, e)
    return out[0, 0]
